# baseline (speedup 1.0000x reference)
import functools

import jax
import jax.numpy as jnp
from jax import lax
from jax.experimental import pallas as pl
from jax.experimental.pallas import tpu as pltpu
from jax.experimental.pallas import tpu_sc as plsc

_NUM_CORES = 2
_NUM_SUBCORES = 16
_NUM_WORKERS = _NUM_CORES * _NUM_SUBCORES

_HIDDEN = 64
_TOKENS_PER_BIN = 20
_GROUP_BINS = 26
_GROUP_ROWS = _GROUP_BINS * _TOKENS_PER_BIN
_IDX_CHUNKS = 5
_IDX_MINOR = _GROUP_ROWS // _IDX_CHUNKS
_H_CHUNKS = _HIDDEN // 16

_GATHER_DNUMS = lax.GatherDimensionNumbers(
    offset_dims=(), collapsed_slice_dims=(0,), start_index_map=(0,))


def _lane_broadcast(vec, i):
  idx = jnp.zeros((16, 1), jnp.int32) + i
  return lax.gather(
      vec, idx, _GATHER_DNUMS, (1,),
      mode=lax.GatherScatterMode.PROMISE_IN_BOUNDS)


def _make_kernel(batch, feats):
  assert batch % _NUM_WORKERS == 0 and feats == _GROUP_BINS
  groups_per_worker = batch // _NUM_WORKERS
  assert groups_per_worker % 2 == 0 and groups_per_worker >= 4

  mesh = plsc.VectorSubcoreMesh(
      core_axis_name="c",
      subcore_axis_name="s",
      num_cores=_NUM_CORES,
      num_subcores=_NUM_SUBCORES,
  )

  @functools.partial(
      pl.kernel,
      mesh=mesh,
      compiler_params=pltpu.CompilerParams(
          needs_layout_passes=False, use_tc_tiling_on_sc=False),
      out_type=jax.ShapeDtypeStruct((batch, feats, _HIDDEN), jnp.float32),
      scratch_types=[
          pltpu.VMEM((_GROUP_ROWS, ), jnp.int32),
          pltpu.VMEM((_GROUP_ROWS, ), jnp.int32),
          pltpu.VMEM((_GROUP_ROWS, _HIDDEN), jnp.float32),
          pltpu.VMEM((_GROUP_ROWS, _HIDDEN), jnp.float32),
          pltpu.VMEM((_GROUP_BINS, _HIDDEN), jnp.float32),
          pltpu.SemaphoreType.DMA,
          pltpu.SemaphoreType.DMA,
          pltpu.SemaphoreType.DMA,
          pltpu.SemaphoreType.DMA,
      ],
  )
  def embed(tok_hbm, table_hbm, out_hbm, tok_v0, tok_v1, rows_v0, rows_v1,
            out_v, tok_sem0, tok_sem1, rows_sem0, rows_sem1):
    wid = lax.axis_index("s") * _NUM_CORES + lax.axis_index("c")
    worker_batch0 = wid * groups_per_worker
    lane = lax.iota(jnp.int32, 16)
    tok_refs = (tok_v0, tok_v1)
    rows_refs = (rows_v0, rows_v1)
    tok_sems = (tok_sem0, tok_sem1)
    rows_sems = (rows_sem0, rows_sem1)

    def fire_tok(k, buf):
      src = tok_hbm.at[pl.ds((worker_batch0 + k) * _GROUP_ROWS, _GROUP_ROWS)]
      pltpu.async_copy(src, tok_refs[buf], tok_sems[buf])

    def drain_tok(buf):
      pltpu.make_async_copy(
          tok_hbm.at[pl.ds(0, _GROUP_ROWS)], tok_refs[buf],
          tok_sems[buf]).wait()

    def fire_gathers(buf):
      for j in range(_IDX_CHUNKS):
        pltpu.async_copy(
            table_hbm.at[tok_refs[buf].at[pl.ds(j * _IDX_MINOR, _IDX_MINOR)]],
            rows_refs[buf].at[pl.ds(j * _IDX_MINOR, _IDX_MINOR)],
            rows_sems[buf],
        )

    def drain_rows(buf):
      pltpu.make_async_copy(
          table_hbm.at[pl.ds(0, _GROUP_ROWS)], rows_refs[buf],
          rows_sems[buf]).wait()

    def phase(g, cur, fire_next_tok, fire_next_gather):
      nxt = 1 - cur
      rows_cur = rows_refs[cur]
      drain_rows(cur)
      if fire_next_gather:
        drain_tok(nxt)
      invs = []
      for h2 in range(2):
        binv = jnp.minimum(lane + h2 * 16, _GROUP_BINS - 1)
        cnt = jnp.zeros((16,), jnp.int32)
        for l in range(_TOKENS_PER_BIN):
          tv = plsc.load_gather(
              tok_refs[cur], [binv * _TOKENS_PER_BIN + l])
          cnt = cnt + jnp.where(tv != 0, jnp.int32(1), jnp.int32(0))
        invs.append(1.0 / jnp.maximum(cnt, 1).astype(jnp.float32))
      if fire_next_tok:
        fire_tok(g + 2, cur)
      if fire_next_gather:
        fire_gathers(nxt)

      def bin_body(b, carry):
        row0 = b * _TOKENS_PER_BIN
        acc = [rows_cur[row0, pl.ds(h * 16, 16)] for h in range(_H_CHUNKS)]
        for l in range(1, _TOKENS_PER_BIN):
          r = row0 + l
          for h in range(_H_CHUNKS):
            acc[h] = acc[h] + rows_cur[r, pl.ds(h * 16, 16)]
        inv = jnp.where(
            b < 16,
            _lane_broadcast(invs[0], jnp.minimum(b, 15)),
            _lane_broadcast(invs[1], jnp.maximum(b - 16, 0)),
        )
        for h in range(_H_CHUNKS):
          out_v[b, pl.ds(h * 16, 16)] = acc[h] * inv
        return carry

      lax.fori_loop(0, _GROUP_BINS, bin_body, 0)

      pltpu.sync_copy(out_v, out_hbm.at[worker_batch0 + g])

    fire_tok(0, 0)
    fire_tok(1, 1)
    drain_tok(0)
    fire_gathers(0)

    def pair_body(g2, carry):
      g = g2 * 2
      phase(g, 0, True, True)
      phase(g + 1, 1, True, True)
      return carry

    lax.fori_loop(0, groups_per_worker // 2 - 1, pair_body, 0)
    phase(groups_per_worker - 2, 0, False, True)
    phase(groups_per_worker - 1, 1, False, False)

  return embed


def kernel(tokens, table):
  assert tokens.ndim == 3 and table.ndim == 2
  batch, feats, tpb = tokens.shape
  assert tpb == _TOKENS_PER_BIN and table.shape[1] == _HIDDEN
  tok_flat = tokens.astype(jnp.int32).reshape(-1)
  return _make_kernel(batch, feats)(tok_flat, table)


# speedup vs baseline: 1.0315x; 1.0315x over previous
import functools

import jax
import jax.numpy as jnp
from jax import lax
from jax.experimental import pallas as pl
from jax.experimental.pallas import tpu as pltpu
from jax.experimental.pallas import tpu_sc as plsc

_NUM_CORES = 2
_NUM_SUBCORES = 16
_NUM_WORKERS = _NUM_CORES * _NUM_SUBCORES

_HIDDEN = 64
_TOKENS_PER_BIN = 20
_GROUP_BINS = 32
_GROUP_ROWS = _GROUP_BINS * _TOKENS_PER_BIN
_IDX_MINOR = 128
_IDX_CHUNKS = _GROUP_ROWS // _IDX_MINOR
_H_CHUNKS = _HIDDEN // 16

_GATHER_DNUMS = lax.GatherDimensionNumbers(
    offset_dims=(), collapsed_slice_dims=(0,), start_index_map=(0,))


def _lane_broadcast(vec, i):
  idx = jnp.zeros((16, 1), jnp.int32) + i
  return lax.gather(
      vec, idx, _GATHER_DNUMS, (1,),
      mode=lax.GatherScatterMode.PROMISE_IN_BOUNDS)


def _make_kernel(num_bins):
  assert num_bins % (_NUM_WORKERS * _GROUP_BINS) == 0
  bins_per_worker = num_bins // _NUM_WORKERS
  num_groups = bins_per_worker // _GROUP_BINS
  assert num_groups % 2 == 0 and num_groups >= 4

  mesh = plsc.VectorSubcoreMesh(
      core_axis_name="c",
      subcore_axis_name="s",
      num_cores=_NUM_CORES,
      num_subcores=_NUM_SUBCORES,
  )

  @functools.partial(
      pl.kernel,
      mesh=mesh,
      compiler_params=pltpu.CompilerParams(
          needs_layout_passes=False, use_tc_tiling_on_sc=False),
      out_type=jax.ShapeDtypeStruct((num_bins * _HIDDEN, ), jnp.float32),
      scratch_types=[
          pltpu.VMEM((_GROUP_ROWS, ), jnp.int32),
          pltpu.VMEM((_GROUP_ROWS, ), jnp.int32),
          pltpu.VMEM((_GROUP_ROWS, _HIDDEN), jnp.float32),
          pltpu.VMEM((_GROUP_ROWS, _HIDDEN), jnp.float32),
          pltpu.VMEM((_GROUP_BINS * _HIDDEN, ), jnp.float32),
          pltpu.VMEM((_GROUP_BINS * _HIDDEN, ), jnp.float32),
          pltpu.SemaphoreType.DMA,
          pltpu.SemaphoreType.DMA,
          pltpu.SemaphoreType.DMA,
          pltpu.SemaphoreType.DMA,
          pltpu.SemaphoreType.DMA,
          pltpu.SemaphoreType.DMA,
      ],
  )
  def embed(tok_hbm, table_hbm, out_hbm, tok_v0, tok_v1, rows_v0, rows_v1,
            out_v0, out_v1, tok_sem0, tok_sem1, rows_sem0, rows_sem1,
            out_sem0, out_sem1):
    wid = lax.axis_index("s") * _NUM_CORES + lax.axis_index("c")
    worker_tok0 = wid * bins_per_worker * _TOKENS_PER_BIN
    worker_out0 = wid * bins_per_worker * _HIDDEN
    lane = lax.iota(jnp.int32, 16)
    tok_refs = (tok_v0, tok_v1)
    rows_refs = (rows_v0, rows_v1)
    out_refs = (out_v0, out_v1)
    tok_sems = (tok_sem0, tok_sem1)
    rows_sems = (rows_sem0, rows_sem1)
    out_sems = (out_sem0, out_sem1)
    out_len = _GROUP_BINS * _HIDDEN

    def fire_tok(k, buf):
      src = tok_hbm.at[pl.ds(worker_tok0 + k * _GROUP_ROWS, _GROUP_ROWS)]
      pltpu.async_copy(src, tok_refs[buf], tok_sems[buf])

    def drain_tok(buf):
      pltpu.make_async_copy(
          tok_hbm.at[pl.ds(0, _GROUP_ROWS)], tok_refs[buf],
          tok_sems[buf]).wait()

    def fire_gathers(buf):
      for j in range(_IDX_CHUNKS):
        pltpu.async_copy(
            table_hbm.at[tok_refs[buf].at[pl.ds(j * _IDX_MINOR, _IDX_MINOR)]],
            rows_refs[buf].at[pl.ds(j * _IDX_MINOR, _IDX_MINOR)],
            rows_sems[buf],
        )

    def drain_rows(buf):
      pltpu.make_async_copy(
          table_hbm.at[pl.ds(0, _GROUP_ROWS)], rows_refs[buf],
          rows_sems[buf]).wait()

    def drain_out(buf):
      pltpu.make_async_copy(
          out_refs[buf], out_hbm.at[pl.ds(0, out_len)],
          out_sems[buf]).wait()

    def phase(g, cur, fire_next_tok, fire_next_gather, drain_prev_out):
      nxt = 1 - cur
      rows_cur = rows_refs[cur]
      out_cur = out_refs[cur]
      drain_rows(cur)
      if drain_prev_out:
        drain_out(cur)
      if fire_next_gather:
        drain_tok(nxt)
      invs = []
      for h2 in range(_GROUP_BINS // 16):
        binv = lane + h2 * 16
        cnt = jnp.zeros((16,), jnp.int32)
        for l in range(_TOKENS_PER_BIN):
          tv = plsc.load_gather(
              tok_refs[cur], [binv * _TOKENS_PER_BIN + l])
          cnt = cnt + jnp.where(tv != 0, jnp.int32(1), jnp.int32(0))
        invs.append(1.0 / jnp.maximum(cnt, 1).astype(jnp.float32))
      if fire_next_tok:
        fire_tok(g + 2, cur)
      if fire_next_gather:
        fire_gathers(nxt)

      def bin_body(b, carry):
        row0 = b * _TOKENS_PER_BIN
        acc = [rows_cur[row0, pl.ds(h * 16, 16)] for h in range(_H_CHUNKS)]
        for l in range(1, _TOKENS_PER_BIN):
          r = row0 + l
          for h in range(_H_CHUNKS):
            acc[h] = acc[h] + rows_cur[r, pl.ds(h * 16, 16)]
        inv = jnp.where(
            b < 16,
            _lane_broadcast(invs[0], jnp.minimum(b, 15)),
            _lane_broadcast(invs[1], jnp.maximum(b - 16, 0)),
        )
        for h in range(_H_CHUNKS):
          out_cur[pl.ds(b * _HIDDEN + h * 16, 16)] = acc[h] * inv
        return carry

      lax.fori_loop(0, _GROUP_BINS, bin_body, 0, unroll=2)

      pltpu.async_copy(
          out_cur,
          out_hbm.at[pl.ds(worker_out0 + g * out_len, out_len)],
          out_sems[cur],
      )

    fire_tok(0, 0)
    fire_tok(1, 1)
    drain_tok(0)
    fire_gathers(0)
    phase(0, 0, True, True, False)
    phase(1, 1, True, True, False)

    def pair_body(g2, carry):
      g = g2 * 2
      phase(g, 0, True, True, True)
      phase(g + 1, 1, True, True, True)
      return carry

    lax.fori_loop(1, num_groups // 2 - 1, pair_body, 0)
    phase(num_groups - 2, 0, False, True, True)
    phase(num_groups - 1, 1, False, False, True)
    drain_out(0)
    drain_out(1)

  return embed


def kernel(tokens, table):
  assert tokens.ndim == 3 and table.ndim == 2
  batch, feats, tpb = tokens.shape
  assert tpb == _TOKENS_PER_BIN and table.shape[1] == _HIDDEN
  num_bins = batch * feats
  tok_flat = tokens.astype(jnp.int32).reshape(-1)
  out = _make_kernel(num_bins)(tok_flat, table)
  return out.reshape(batch, feats, _HIDDEN)


# speedup vs baseline: 1.0327x; 1.0012x over previous
import functools

import jax
import jax.numpy as jnp
from jax import lax
from jax.experimental import pallas as pl
from jax.experimental.pallas import tpu as pltpu
from jax.experimental.pallas import tpu_sc as plsc

_NUM_CORES = 2
_NUM_SUBCORES = 16
_NUM_WORKERS = _NUM_CORES * _NUM_SUBCORES

_HIDDEN = 64
_TOKENS_PER_BIN = 20
_GROUP_BINS = 32
_GROUP_ROWS = _GROUP_BINS * _TOKENS_PER_BIN
_IDX_MINOR = 128
_IDX_CHUNKS = _GROUP_ROWS // _IDX_MINOR
_H_CHUNKS = _HIDDEN // 16

_GATHER_DNUMS = lax.GatherDimensionNumbers(
    offset_dims=(), collapsed_slice_dims=(0,), start_index_map=(0,))


def _lane_broadcast(vec, i):
  idx = jnp.zeros((16, 1), jnp.int32) + i
  return lax.gather(
      vec, idx, _GATHER_DNUMS, (1,),
      mode=lax.GatherScatterMode.PROMISE_IN_BOUNDS)


def _make_kernel(num_bins):
  assert num_bins % (_NUM_WORKERS * _GROUP_BINS) == 0
  bins_per_worker = num_bins // _NUM_WORKERS
  num_groups = bins_per_worker // _GROUP_BINS
  assert num_groups % 2 == 0 and num_groups >= 4

  mesh = plsc.VectorSubcoreMesh(
      core_axis_name="c",
      subcore_axis_name="s",
      num_cores=_NUM_CORES,
      num_subcores=_NUM_SUBCORES,
  )

  @functools.partial(
      pl.kernel,
      mesh=mesh,
      compiler_params=pltpu.CompilerParams(
          needs_layout_passes=False, use_tc_tiling_on_sc=False),
      out_type=jax.ShapeDtypeStruct((num_bins * _HIDDEN, ), jnp.float32),
      scratch_types=[
          pltpu.VMEM((_GROUP_ROWS, ), jnp.int32),
          pltpu.VMEM((_GROUP_ROWS, ), jnp.int32),
          pltpu.VMEM((_GROUP_ROWS, _HIDDEN), jnp.float32),
          pltpu.VMEM((_GROUP_ROWS, _HIDDEN), jnp.float32),
          pltpu.VMEM((_GROUP_BINS * _HIDDEN, ), jnp.float32),
          pltpu.VMEM((_GROUP_BINS * _HIDDEN, ), jnp.float32),
          pltpu.SemaphoreType.DMA,
          pltpu.SemaphoreType.DMA,
          pltpu.SemaphoreType.DMA,
          pltpu.SemaphoreType.DMA,
          pltpu.SemaphoreType.DMA,
          pltpu.SemaphoreType.DMA,
      ],
  )
  def embed(tok_hbm, table_hbm, out_hbm, tok_v0, tok_v1, rows_v0, rows_v1,
            out_v0, out_v1, tok_sem0, tok_sem1, rows_sem0, rows_sem1,
            out_sem0, out_sem1):
    wid = lax.axis_index("s") * _NUM_CORES + lax.axis_index("c")
    worker_tok0 = wid * bins_per_worker * _TOKENS_PER_BIN
    worker_out0 = wid * bins_per_worker * _HIDDEN
    lane = lax.iota(jnp.int32, 16)
    tok_refs = (tok_v0, tok_v1)
    rows_refs = (rows_v0, rows_v1)
    out_refs = (out_v0, out_v1)
    tok_sems = (tok_sem0, tok_sem1)
    rows_sems = (rows_sem0, rows_sem1)
    out_sems = (out_sem0, out_sem1)
    out_len = _GROUP_BINS * _HIDDEN

    def fire_tok(k, buf):
      src = tok_hbm.at[pl.ds(worker_tok0 + k * _GROUP_ROWS, _GROUP_ROWS)]
      pltpu.async_copy(src, tok_refs[buf], tok_sems[buf])

    def drain_tok(buf):
      pltpu.make_async_copy(
          tok_hbm.at[pl.ds(0, _GROUP_ROWS)], tok_refs[buf],
          tok_sems[buf]).wait()

    def fire_gathers(buf):
      for j in range(_IDX_CHUNKS):
        pltpu.async_copy(
            table_hbm.at[tok_refs[buf].at[pl.ds(j * _IDX_MINOR, _IDX_MINOR)]],
            rows_refs[buf].at[pl.ds(j * _IDX_MINOR, _IDX_MINOR)],
            rows_sems[buf],
        )

    def drain_rows(buf):
      pltpu.make_async_copy(
          table_hbm.at[pl.ds(0, _GROUP_ROWS)], rows_refs[buf],
          rows_sems[buf]).wait()

    def drain_out(buf):
      pltpu.make_async_copy(
          out_refs[buf], out_hbm.at[pl.ds(0, out_len)],
          out_sems[buf]).wait()

    def phase(g, cur, fire_next_tok, fire_next_gather, drain_prev_out):
      nxt = 1 - cur
      rows_cur = rows_refs[cur]
      out_cur = out_refs[cur]
      drain_rows(cur)
      if drain_prev_out:
        drain_out(cur)
      if fire_next_gather:
        drain_tok(nxt)
      invs = []
      for h2 in range(_GROUP_BINS // 16):
        binv = lane + h2 * 16
        cnt = jnp.zeros((16,), jnp.int32)
        for l in range(_TOKENS_PER_BIN):
          tv = plsc.load_gather(
              tok_refs[cur], [binv * _TOKENS_PER_BIN + l])
          cnt = cnt + jnp.where(tv != 0, jnp.int32(1), jnp.int32(0))
        invs.append(1.0 / jnp.maximum(cnt, 1).astype(jnp.float32))
      if fire_next_tok:
        fire_tok(g + 2, cur)
      if fire_next_gather:
        fire_gathers(nxt)

      def bin_body(b, carry):
        row0 = b * _TOKENS_PER_BIN
        acc = [rows_cur[row0, pl.ds(h * 16, 16)] for h in range(_H_CHUNKS)]
        for l in range(1, _TOKENS_PER_BIN):
          r = row0 + l
          for h in range(_H_CHUNKS):
            acc[h] = acc[h] + rows_cur[r, pl.ds(h * 16, 16)]
        inv = jnp.where(
            b < 16,
            _lane_broadcast(invs[0], jnp.minimum(b, 15)),
            _lane_broadcast(invs[1], jnp.maximum(b - 16, 0)),
        )
        for h in range(_H_CHUNKS):
          out_cur[pl.ds(b * _HIDDEN + h * 16, 16)] = acc[h] * inv
        return carry

      lax.fori_loop(0, _GROUP_BINS, bin_body, 0, unroll=4)

      pltpu.async_copy(
          out_cur,
          out_hbm.at[pl.ds(worker_out0 + g * out_len, out_len)],
          out_sems[cur],
      )

    fire_tok(0, 0)
    fire_tok(1, 1)
    drain_tok(0)
    fire_gathers(0)
    phase(0, 0, True, True, False)
    phase(1, 1, True, True, False)

    def pair_body(g2, carry):
      g = g2 * 2
      phase(g, 0, True, True, True)
      phase(g + 1, 1, True, True, True)
      return carry

    lax.fori_loop(1, num_groups // 2 - 1, pair_body, 0)
    phase(num_groups - 2, 0, False, True, True)
    phase(num_groups - 1, 1, False, False, True)
    drain_out(0)
    drain_out(1)

  return embed


def kernel(tokens, table):
  assert tokens.ndim == 3 and table.ndim == 2
  batch, feats, tpb = tokens.shape
  assert tpb == _TOKENS_PER_BIN and table.shape[1] == _HIDDEN
  num_bins = batch * feats
  tok_flat = tokens.astype(jnp.int32).reshape(-1)
  out = _make_kernel(num_bins)(tok_flat, table)
  return out.reshape(batch, feats, _HIDDEN)
